# Initial kernel scaffold; baseline (speedup 1.0000x reference)
import jax
import jax.numpy as jnp
from jax.experimental import pallas as pl


def kernel(users, pos_items, neg_items, content_emb, enc_W1, enc_b1, enc_W2, enc_b2, enc_W3, enc_b3, codebooks, dec_W1, dec_b1, dec_W2, dec_b2, dec_W3, dec_b3, user_emb, item_emb, edge_u, edge_i):
    raise NotImplementedError("write your pallas kernel here")


# speedup vs baseline: 1.0079x; 1.0079x over previous
import functools
import jax
import jax.numpy as jnp
from jax.experimental import pallas as pl
from jax.experimental.pallas import tpu as pltpu

N_USERS = 20000
N_ITEMS = 20000
EMB_DIM = 768
E_DIM = 256
H1, H2 = 512, 256
N_LEVELS = 3
K = 256
GCN_DIM = 256
GCN_LAYERS = 2
N_EDGES = 320000
B = 4096
BETA = 0.25
QUANT_W = 1.0
WEIGHT_DECAY = 1e-4
LAMBDA_ALIGN = 0.5

BM = 2000
NB = N_ITEMS // BM


def _rqvae_body(x_ref, eW1, eb1, eW2, eb2, eW3, eb3, cb_ref,
                dW1, db1, dW2, db2, dW3, db3,
                quant_ref, loss_ref):
    i = pl.program_id(0)

    @pl.when(i == 0)
    def _init():
        loss_ref[...] = jnp.zeros_like(loss_ref)

    x = x_ref[...]
    h = jnp.maximum(x @ eW1[...] + eb1[...], 0.0)
    h = jnp.maximum(h @ eW2[...] + eb2[...], 0.0)
    z = h @ eW3[...] + eb3[...]

    residual = z
    quantized = jnp.zeros_like(z)
    quant_sse = jnp.float32(0.0)
    iota_k = jax.lax.broadcasted_iota(jnp.int32, (BM, K), 1)
    for l in range(N_LEVELS):
        cb = cb_ref[l]
        d = (jnp.sum(residual * residual, axis=1, keepdims=True)
             - 2.0 * jax.lax.dot_general(residual, cb, (((1,), (1,)), ((), ())))
             + jnp.sum(cb * cb, axis=1)[None, :])
        min_d = jnp.min(d, axis=1, keepdims=True)
        masked_iota = jnp.where(d <= min_d, iota_k, K)
        idx = jnp.min(masked_iota, axis=1)
        onehot = (iota_k == idx[:, None]).astype(jnp.float32)
        e = onehot @ cb
        diff = residual - e
        quant_sse = quant_sse + jnp.sum(diff * diff)
        quantized = quantized + e
        residual = diff

    quant_ref[...] = quantized

    h = jnp.maximum(quantized @ dW1[...] + db1[...], 0.0)
    h = jnp.maximum(h @ dW2[...] + db2[...], 0.0)
    rec = h @ dW3[...] + db3[...]
    rdiff = rec - x
    recon_sse = jnp.sum(rdiff * rdiff)

    lane = jax.lax.broadcasted_iota(jnp.int32, (1, 128), 1)
    upd = (jnp.where(lane == 0, recon_sse, 0.0)
           + jnp.where(lane == 1, quant_sse, 0.0))
    loss_ref[...] = loss_ref[...] + upd


def _rqvae(content_emb, enc_W1, enc_b1, enc_W2, enc_b2, enc_W3, enc_b3,
           codebooks, dec_W1, dec_b1, dec_W2, dec_b2, dec_W3, dec_b3):
    full = lambda shape: pl.BlockSpec(shape, lambda i: (0,) * len(shape))
    quantized, losses = pl.pallas_call(
        _rqvae_body,
        grid=(NB,),
        in_specs=[
            pl.BlockSpec((BM, EMB_DIM), lambda i: (i, 0)),
            full((EMB_DIM, H1)), full((1, H1)),
            full((H1, H2)), full((1, H2)),
            full((H2, E_DIM)), full((1, E_DIM)),
            full((N_LEVELS, K, E_DIM)),
            full((E_DIM, H2)), full((1, H2)),
            full((H2, H1)), full((1, H1)),
            full((H1, EMB_DIM)), full((1, EMB_DIM)),
        ],
        out_specs=[
            pl.BlockSpec((BM, E_DIM), lambda i: (i, 0)),
            pl.BlockSpec((1, 128), lambda i: (0, 0)),
        ],
        out_shape=[
            jax.ShapeDtypeStruct((N_ITEMS, E_DIM), jnp.float32),
            jax.ShapeDtypeStruct((1, 128), jnp.float32),
        ],
    )(content_emb, enc_W1, enc_b1.reshape(1, -1), enc_W2, enc_b2.reshape(1, -1),
      enc_W3, enc_b3.reshape(1, -1), codebooks,
      dec_W1, dec_b1.reshape(1, -1), dec_W2, dec_b2.reshape(1, -1),
      dec_W3, dec_b3.reshape(1, -1))
    loss_recon = losses[0, 0] / (N_ITEMS * EMB_DIM)
    quant_loss = (1.0 + BETA) * losses[0, 1] / (N_ITEMS * E_DIM)
    return quantized, loss_recon, quant_loss


def kernel(users, pos_items, neg_items, content_emb,
           enc_W1, enc_b1, enc_W2, enc_b2, enc_W3, enc_b3,
           codebooks, dec_W1, dec_b1, dec_W2, dec_b2, dec_W3, dec_b3,
           user_emb, item_emb, edge_u, edge_i):
    quantized, loss_recon, quant_loss = _rqvae(
        content_emb, enc_W1, enc_b1, enc_W2, enc_b2, enc_W3, enc_b3,
        codebooks, dec_W1, dec_b1, dec_W2, dec_b2, dec_W3, dec_b3)

    deg_u = jnp.zeros((N_USERS,), dtype=jnp.float32).at[edge_u].add(1.0)
    deg_i = jnp.zeros((N_ITEMS,), dtype=jnp.float32).at[edge_i].add(1.0)
    norm = 1.0 / jnp.sqrt(jnp.clip(deg_u[edge_u] * deg_i[edge_i], 1.0))
    u, it = user_emb, item_emb
    u_acc, i_acc = user_emb, item_emb
    for _ in range(GCN_LAYERS):
        new_u = jnp.zeros_like(u).at[edge_u].add(norm[:, None] * jnp.take(it, edge_i, axis=0))
        new_i = jnp.zeros_like(it).at[edge_i].add(norm[:, None] * jnp.take(u, edge_u, axis=0))
        u, it = new_u, new_i
        u_acc = u_acc + u
        i_acc = i_acc + it
    u_final = u_acc / (GCN_LAYERS + 1)
    i_final = i_acc / (GCN_LAYERS + 1)

    u_e = jnp.take(u_final, users, axis=0)
    p_e = jnp.take(i_final, pos_items, axis=0)
    n_e = jnp.take(i_final, neg_items, axis=0)
    pos_scores = jnp.sum(u_e * p_e, axis=1)
    neg_scores = jnp.sum(u_e * n_e, axis=1)
    cf_bpr_loss = jnp.mean(jax.nn.softplus(neg_scores - pos_scores))
    cf_reg_loss = 0.5 * WEIGHT_DECAY * (
        jnp.sum(jnp.take(user_emb, users, axis=0) ** 2)
        + jnp.sum(jnp.take(item_emb, pos_items, axis=0) ** 2)
        + jnp.sum(jnp.take(item_emb, neg_items, axis=0) ** 2)) / users.shape[0]
    pos_q = jnp.take(quantized, pos_items, axis=0)
    pos_cf = jnp.take(item_emb, pos_items, axis=0)
    align_loss = jnp.mean(jnp.abs(pos_cf - pos_q))
    total_loss = (loss_recon + QUANT_W * quant_loss + cf_bpr_loss
                  + cf_reg_loss + LAMBDA_ALIGN * align_loss)
    return total_loss
